# baseline (speedup 1.0000x reference)
import jax
import jax.numpy as jnp
from jax.experimental import pallas as pl
from jax.experimental.pallas import tpu as pltpu

K_DET = 64


def _body(hm_ref, reg_ref, wh_ref, conv_ref, bb_ref, cv_ref, nms_ref):
    C, H, W = hm_ref.shape[1], hm_ref.shape[2], hm_ref.shape[3]
    CW = conv_ref.shape[1]
    x = hm_ref[0]
    s = jax.nn.sigmoid(x)
    ninf = jnp.float32(-jnp.inf)
    padh = jnp.full((C, 1, W), ninf, jnp.float32)
    v = jnp.maximum(s, jnp.concatenate([s[:, 1:, :], padh], axis=1))
    v = jnp.maximum(v, jnp.concatenate([padh, s[:, :-1, :]], axis=1))
    padw = jnp.full((C, H, 1), ninf, jnp.float32)
    hmax = jnp.maximum(v, jnp.concatenate([v[:, :, 1:], padw], axis=2))
    hmax = jnp.maximum(hmax, jnp.concatenate([padw, v[:, :, :-1]], axis=2))
    nmsed = jnp.where(s == hmax, s, jnp.float32(0.0))
    nms_ref[...] = nmsed
    l1_0 = jnp.max(nmsed, axis=2)

    flat_ci = (jax.lax.broadcasted_iota(jnp.int32, (C, H), 0) * H
               + jax.lax.broadcasted_iota(jnp.int32, (C, H), 1))
    jiota = jax.lax.broadcasted_iota(jnp.int32, (1, W), 1)
    kiota = jax.lax.broadcasted_iota(jnp.int32, (1, K_DET), 1)
    krows = jax.lax.broadcasted_iota(jnp.int32, (K_DET, 1), 0)
    big = jnp.int32(1 << 30)

    def step(k, carry):
        l1, gvec, vvec, gcol = carry
        mv = jnp.max(l1, keepdims=True)
        f = jnp.min(jnp.where(l1 == mv, flat_ci, big))
        c = jax.lax.shift_right_logical(f, 7)
        i = jax.lax.bitwise_and(f, jnp.int32(H - 1))
        row = nms_ref[c, pl.ds(i, 1), :]
        jm = jnp.min(jnp.where(row == mv, jiota, big), keepdims=True)
        newrow = jnp.where(jiota == jm, jnp.float32(-1.0), row)
        nms_ref[c, pl.ds(i, 1), :] = newrow
        l1 = jnp.where(flat_ci == f, jnp.max(newrow, keepdims=True), l1)
        gb = f * W + jm
        sel = kiota == k
        gvec = jnp.where(sel, gb, gvec)
        vvec = jnp.where(sel, mv, vvec)
        gcol = jnp.where(krows == k, gb, gcol)
        return (l1, gvec, vvec, gcol)

    carry0 = (l1_0,
              jnp.zeros((1, K_DET), jnp.int32),
              jnp.zeros((1, K_DET), jnp.float32),
              jnp.zeros((K_DET, 1), jnp.int32))
    _, gvec, vvec, gcol = jax.lax.fori_loop(0, K_DET, step, carry0)

    z = jnp.zeros((1, K_DET), jnp.float32)
    r0a = r1a = w0a = w1a = z
    for k in range(K_DET):
        g = jnp.sum(jnp.where(kiota == k, gvec, 0))
        i = jax.lax.bitwise_and(jax.lax.shift_right_logical(g, 7),
                                jnp.int32(H - 1))
        j = jax.lax.bitwise_and(g, jnp.int32(W - 1))
        sh = jnp.int32(W) - j
        sel = kiota == k
        r0a = jnp.where(sel,
                        pltpu.roll(reg_ref[0, 0, pl.ds(i, 1), :], sh, 1)[:, 0:1],
                        r0a)
        r1a = jnp.where(sel,
                        pltpu.roll(reg_ref[0, 1, pl.ds(i, 1), :], sh, 1)[:, 0:1],
                        r1a)
        w0a = jnp.where(sel,
                        pltpu.roll(wh_ref[0, 0, pl.ds(i, 1), :], sh, 1)[:, 0:1],
                        w0a)
        w1a = jnp.where(sel,
                        pltpu.roll(wh_ref[0, 1, pl.ds(i, 1), :], sh, 1)[:, 0:1],
                        w1a)

    pcol = jax.lax.bitwise_and(gcol, jnp.int32(H * W - 1))
    onehot = (jax.lax.broadcasted_iota(jnp.int32, (K_DET, H * W), 1)
              == pcol).astype(jnp.float32)
    convr = conv_ref[0].reshape(CW, H * W)
    cv_ref[0] = jax.lax.dot_general(onehot, convr, (((1,), (1,)), ((), ())),
                                    precision=jax.lax.Precision.HIGHEST)

    iv = jax.lax.bitwise_and(jax.lax.shift_right_logical(gvec, 7),
                             jnp.int32(H - 1))
    jv = jax.lax.bitwise_and(gvec, jnp.int32(W - 1))
    cf = jax.lax.shift_right_logical(gvec, 14).astype(jnp.float32)
    xc = jv.astype(jnp.float32) + r0a
    yc = iv.astype(jnp.float32) + r1a
    half = jnp.float32(0.5)
    bb_ref[0] = jnp.concatenate(
        [xc - w0a * half, yc - w1a * half, xc + w0a * half, yc + w1a * half,
         vvec, cf, jnp.zeros((2, K_DET), jnp.float32)],
        axis=0)


def kernel(hm, reg, wh, seg_feat, conv_weight):
    B, C, H, W = hm.shape
    CW = conv_weight.shape[1]
    bb, cv = pl.pallas_call(
        _body,
        grid=(B,),
        in_specs=[
            pl.BlockSpec((1, C, H, W), lambda b: (b, 0, 0, 0)),
            pl.BlockSpec((1, 2, H, W), lambda b: (b, 0, 0, 0)),
            pl.BlockSpec((1, 2, H, W), lambda b: (b, 0, 0, 0)),
            pl.BlockSpec((1, CW, H, W), lambda b: (b, 0, 0, 0)),
        ],
        out_specs=[
            pl.BlockSpec((1, 8, K_DET), lambda b: (b, 0, 0)),
            pl.BlockSpec((1, K_DET, CW), lambda b: (b, 0, 0)),
        ],
        out_shape=[
            jax.ShapeDtypeStruct((B, 8, K_DET), jnp.float32),
            jax.ShapeDtypeStruct((B, K_DET, CW), jnp.float32),
        ],
        scratch_shapes=[
            pltpu.VMEM((C, H, W), jnp.float32),
        ],
    )(hm, reg, wh, conv_weight)
    bboxes = jnp.transpose(bb[:, 0:6, :], (0, 2, 1))
    return (bboxes, seg_feat, cv)
